# Initial kernel scaffold; baseline (speedup 1.0000x reference)
import functools

import jax
import jax.numpy as jnp
from jax import lax
from jax.experimental import pallas as pl
from jax.experimental.pallas import tpu as pltpu
from jax.experimental.pallas import tpu_sc as plsc

_NU = 25000
_NTOT = 50000
_D = 64
_NE = 800000
_B = 4096
_NSLOT = 2 * _B
_L = 16
_NS = 16
_NC = 2
_NW = _NC * _NS

_NVEC = _NE // _L
_VEC_LO = _NVEC // _NW
_VEC_HI = _VEC_LO + 1
_N_HI = _NVEC - _NW * _VEC_LO
_CHUNK_V = 128
_CHUNK_E = _CHUNK_V * _L
_N_CHUNKS = (_VEC_HI + _CHUNK_V - 1) // _CHUNK_V
_CMAX = ((_VEC_HI * _L + 127) // 128) * 128
_NB_MAX = _CMAX // 128
_DUMMY = _NSLOT
_ACC_ROWS = _NSLOT + 128
_ZROWS = _ACC_ROWS // _NS
_JPT = _NSLOT // _NS


def _make_sc_kernel():
    mesh = plsc.VectorSubcoreMesh(core_axis_name="c", subcore_axis_name="s")

    @functools.partial(
        pl.kernel,
        out_type=jax.ShapeDtypeStruct((_NC, _NSLOT, _D), jnp.float32),
        mesh=mesh,
        scratch_types=[
            pltpu.VMEM((_NTOT,), jnp.int32),
            pltpu.VMEM((_NSLOT,), jnp.int32),
            pltpu.VMEM((_CHUNK_E,), jnp.int32),
            pltpu.VMEM((_CHUNK_E,), jnp.int32),
            pltpu.VMEM((_NB_MAX, 128), jnp.int32),
            pltpu.VMEM((_NB_MAX, 128), jnp.int32),
            pltpu.VMEM((128, _D), jnp.float32),
            pltpu.VMEM((_JPT,), jnp.int32),
            pltpu.VMEM((_L,), jnp.int32),
            pltpu.VMEM_SHARED((_ACC_ROWS, _D), jnp.float32),
            pltpu.SemaphoreType.DMA,
        ],
    )
    def sc_fn(edges, needed, feat, zeros2d, neg1, out,
              map_ref, nbuf, dstbuf, srcbuf, csrc, cslot, rows, slotblk,
              cnt_ref, acc, sem):
        cid = lax.axis_index("c")
        sid = lax.axis_index("s")
        wid = cid * _NS + sid

        iota = lax.iota(jnp.int32, _L)

        zbase = pl.multiple_of(sid * _ZROWS, 8)
        pltpu.sync_copy(zeros2d.at[pl.ds(0, _ZROWS)],
                        acc.at[pl.ds(zbase, _ZROWS)])

        pltpu.sync_copy(needed, nbuf)
        pltpu.sync_copy(neg1, map_ref)

        def _map_body(i, carry):
            for u in range(4):
                off = pl.multiple_of(i * 64 + u * 16, 16)
                vals = nbuf[pl.ds(off, _L)]
                plsc.store_scatter(map_ref, [vals], iota + off)
            return carry

        lax.fori_loop(0, _NSLOT // (4 * _L), _map_body, 0)

        plsc.subcore_barrier()

        base_vec = jnp.where(wid < _N_HI, wid * _VEC_HI,
                             _N_HI * _VEC_HI + (wid - _N_HI) * _VEC_LO)
        n_vec = jnp.where(wid < _N_HI, _VEC_HI, _VEC_LO)

        def _filter_vec(voff, cnt):
            off = pl.multiple_of(voff * _L, 16)
            d = dstbuf[pl.ds(off, _L)]
            s = srcbuf[pl.ds(off, _L)]
            slot = plsc.load_gather(map_ref, [d])
            m = slot >= 0
            pos = cnt + plsc.cumsum(m.astype(jnp.int32)) - 1
            row = lax.shift_right_logical(pos, 7)
            col = lax.bitwise_and(pos, 127)
            plsc.store_scatter(csrc, [row, col], s, mask=m)
            plsc.store_scatter(cslot, [row, col], slot, mask=m)
            return cnt + plsc.all_reduce_population_count(m)

        cnt = jnp.zeros((_L,), jnp.int32)
        for k in range(_N_CHUNKS):
            chunk_lo = base_vec * _L + k * _CHUNK_E
            dma_base = pl.multiple_of(
                jnp.minimum(chunk_lo, _NE - _CHUNK_E), 8)
            off_vec = lax.shift_right_logical(chunk_lo - dma_base, 4)
            nv = jnp.clip(n_vec - k * _CHUNK_V, 0, _CHUNK_V)
            pltpu.sync_copy(edges.at[1, pl.ds(dma_base, _CHUNK_E)], dstbuf)
            pltpu.sync_copy(edges.at[0, pl.ds(dma_base, _CHUNK_E)], srcbuf)
            if k < _N_CHUNKS - 1:
                def _quad(i, c):
                    for u in range(4):
                        c = _filter_vec(i * 4 + u, c)
                    return c
                cnt = lax.fori_loop(0, _CHUNK_V // 4, _quad, cnt)
            else:
                def _one(i, c):
                    return _filter_vec(off_vec + i, c)
                cnt = lax.fori_loop(0, nv, _one, cnt)

        cnt_ref[...] = cnt
        n = cnt_ref[0]
        n_pad = lax.bitwise_and(n + 127, ~127)

        def _pad_vec(v, carry):
            pos = iota + v * _L
            mpad = pos >= n
            row = lax.shift_right_logical(pos, 7)
            col = lax.bitwise_and(pos, 127)
            plsc.store_scatter(csrc, [row, col],
                               jnp.zeros((_L,), jnp.int32), mask=mpad)
            plsc.store_scatter(cslot, [row, col],
                               jnp.full((_L,), _DUMMY, jnp.int32), mask=mpad)
            return carry

        lax.fori_loop(lax.shift_right_logical(n, 4),
                      lax.shift_right_logical(n_pad, 4), _pad_vec, 0)

        def _block(j, carry):
            pltpu.async_copy(feat.at[csrc.at[j]], rows, sem).wait()
            pltpu.sync_copy(rows, acc.at[cslot.at[j]], add=True)
            return carry

        lax.fori_loop(0, lax.shift_right_logical(n_pad, 7), _block, 0)

        plsc.subcore_barrier()

        jbase = pl.multiple_of(sid * _JPT, 16)
        for vb in range(_JPT // 128):
            for v in range(8):
                off = pl.multiple_of(jbase + vb * 128 + v * 16, 16)
                vals = nbuf[pl.ds(off, _L)]
                sl = plsc.load_gather(map_ref, [vals])
                slotblk[pl.ds(vb * 128 + v * 16, _L)] = sl
            pltpu.async_copy(acc.at[slotblk.at[pl.ds(vb * 128, 128)]],
                             rows, sem).wait()
            pltpu.sync_copy(rows, out.at[cid, pl.ds(jbase + vb * 128, 128)])

    return sc_fn


def _tc_body(acc_ref, w_ref, b_ref, o_ref):
    a = acc_ref[0] + acc_ref[1]
    p = lax.dot_general(a, w_ref[...], (((1,), (1,)), ((), ())),
                        preferred_element_type=jnp.float32)
    p = p + b_ref[...]
    u = p[:_B]
    t = p[_B:]
    o_ref[...] = jnp.sum(u * t, axis=1, keepdims=True)


def kernel(user_indices, item_indices, edge_index, user_table, item_table,
           W, b):
    needed = jnp.concatenate([user_indices, item_indices + _NU])
    feat = jnp.concatenate([user_table, item_table], axis=0)
    zeros2d = jnp.zeros((_ZROWS, _D), jnp.float32)
    neg1 = jnp.full((_NTOT,), -1, jnp.int32)

    sc_fn = _make_sc_kernel()
    partials = sc_fn(edge_index, needed, feat, zeros2d, neg1)

    out = pl.pallas_call(
        _tc_body,
        out_shape=jax.ShapeDtypeStruct((_B, 1), jnp.float32),
    )(partials, W, jnp.reshape(b, (1, _D)))
    return out


# speedup vs baseline: 17.9033x; 17.9033x over previous
import functools

import jax
import jax.numpy as jnp
from jax import lax
from jax.experimental import pallas as pl
from jax.experimental.pallas import tpu as pltpu
from jax.experimental.pallas import tpu_sc as plsc

_NU = 25000
_NTOT = 50000
_D = 64
_NE = 800000
_B = 4096
_NSLOT = 2 * _B
_L = 16
_NS = 16
_NC = 2
_NW = _NC * _NS

_NVEC = _NE // _L
_VEC_LO = _NVEC // _NW
_VEC_HI = _VEC_LO + 1
_N_HI = _NVEC - _NW * _VEC_LO
_CHUNK_V = 128
_CHUNK_E = _CHUNK_V * _L
_N_CHUNKS = (_VEC_HI + _CHUNK_V - 1) // _CHUNK_V
_CMAX = ((_VEC_HI * _L + 127) // 128) * 128
_NB_MAX = _CMAX // 128
_DUMMY = _NSLOT
_ACC_ROWS = _NSLOT + 128
_ZROWS = _ACC_ROWS // _NS
_JPT = _NSLOT // _NS


def _make_sc_kernel():
    mesh = plsc.VectorSubcoreMesh(core_axis_name="c", subcore_axis_name="s")

    @functools.partial(
        pl.kernel,
        out_type=jax.ShapeDtypeStruct((_NC, _NSLOT, _D), jnp.float32),
        mesh=mesh,
        scratch_types=[
            pltpu.VMEM((_NTOT,), jnp.int32),
            pltpu.VMEM((_NSLOT,), jnp.int32),
            pltpu.VMEM((_CHUNK_E,), jnp.int32),
            pltpu.VMEM((_CHUNK_E,), jnp.int32),
            pltpu.VMEM((_CMAX,), jnp.int32),
            pltpu.VMEM((128,), jnp.int32),
            pltpu.VMEM((128,), jnp.int32),
            pltpu.VMEM((128, _D), jnp.float32),
            pltpu.VMEM((_JPT,), jnp.int32),
            pltpu.VMEM((_L,), jnp.int32),
            pltpu.VMEM_SHARED((_ACC_ROWS, _D), jnp.float32),
            pltpu.SemaphoreType.DMA,
        ],
        compiler_params=pltpu.CompilerParams(needs_layout_passes=False,
                                             use_tc_tiling_on_sc=False),
    )
    def sc_fn(esrc, edst, needed, feat, zeros2d, neg1, out,
              map_ref, nbuf, dstbuf, srcbuf, cpk, sstage, tstage, rows,
              slotblk, cnt_ref, acc, sem):
        cid = lax.axis_index("c")
        sid = lax.axis_index("s")
        wid = cid * _NS + sid

        iota = lax.iota(jnp.int32, _L)

        zbase = pl.multiple_of(sid * _ZROWS, 8)
        pltpu.sync_copy(zeros2d.at[pl.ds(0, _ZROWS)],
                        acc.at[pl.ds(zbase, _ZROWS)])

        pltpu.sync_copy(needed, nbuf)
        pltpu.sync_copy(neg1, map_ref)

        def _map_body(i, carry):
            for u in range(4):
                off = pl.multiple_of(i * 64 + u * 16, 16)
                vals = nbuf[pl.ds(off, _L)]
                plsc.store_scatter(map_ref, [vals], iota + off)
            return carry

        lax.fori_loop(0, _NSLOT // (4 * _L), _map_body, 0)

        plsc.subcore_barrier()

        base_vec = jnp.where(wid < _N_HI, wid * _VEC_HI,
                             _N_HI * _VEC_HI + (wid - _N_HI) * _VEC_LO)
        n_vec = jnp.where(wid < _N_HI, _VEC_HI, _VEC_LO)

        def _filter_vec(voff, cnt):
            off = pl.multiple_of(voff * _L, 16)
            d = dstbuf[pl.ds(off, _L)]
            s = srcbuf[pl.ds(off, _L)]
            slot = plsc.load_gather(map_ref, [d])
            m = slot >= 0
            pos = cnt + plsc.cumsum(m.astype(jnp.int32)) - 1
            packed = lax.bitwise_or(s, lax.shift_left(slot, 17))
            plsc.store_scatter(cpk, [pos], packed, mask=m)
            return cnt + plsc.all_reduce_population_count(m)

        cnt = jnp.zeros((_L,), jnp.int32)
        for k in range(_N_CHUNKS):
            chunk_lo = base_vec * _L + k * _CHUNK_E
            dma_base = pl.multiple_of(
                jnp.minimum(chunk_lo, _NE - _CHUNK_E), 8)
            off_vec = lax.shift_right_logical(chunk_lo - dma_base, 4)
            nv = jnp.clip(n_vec - k * _CHUNK_V, 0, _CHUNK_V)
            pltpu.sync_copy(edst.at[pl.ds(dma_base, _CHUNK_E)], dstbuf)
            pltpu.sync_copy(esrc.at[pl.ds(dma_base, _CHUNK_E)], srcbuf)
            if k < _N_CHUNKS - 1:
                def _quad(i, c):
                    for u in range(4):
                        c = _filter_vec(i * 4 + u, c)
                    return c
                cnt = lax.fori_loop(0, _CHUNK_V // 4, _quad, cnt)
            else:
                def _one(i, c):
                    return _filter_vec(off_vec + i, c)
                cnt = lax.fori_loop(0, nv, _one, cnt)

        cnt_ref[...] = cnt
        n = cnt_ref[...][0]
        n_pad = lax.bitwise_and(n + 127, ~127)

        def _pad_vec(v, carry):
            pos = iota + v * _L
            mpad = pos >= n
            plsc.store_scatter(cpk, [pos],
                               jnp.full((_L,), _DUMMY << 17, jnp.int32),
                               mask=mpad)
            return carry

        lax.fori_loop(lax.shift_right_logical(n, 4),
                      lax.shift_right_logical(n_pad, 4), _pad_vec, 0)

        def _block(j, carry):
            for v in range(8):
                off = pl.multiple_of(j * 128 + v * 16, 16)
                w = cpk[pl.ds(off, _L)]
                sstage[pl.ds(v * 16, _L)] = lax.bitwise_and(w, (1 << 17) - 1)
                tstage[pl.ds(v * 16, _L)] = lax.shift_right_logical(w, 17)
            pltpu.async_copy(feat.at[sstage], rows, sem).wait()
            pltpu.sync_copy(rows, acc.at[tstage], add=True)
            return carry

        lax.fori_loop(0, lax.shift_right_logical(n_pad, 7), _block, 0)

        plsc.subcore_barrier()

        jbase = pl.multiple_of(sid * _JPT, 16)
        for vb in range(_JPT // 128):
            for v in range(8):
                off = pl.multiple_of(jbase + vb * 128 + v * 16, 16)
                vals = nbuf[pl.ds(off, _L)]
                sl = plsc.load_gather(map_ref, [vals])
                slotblk[pl.ds(vb * 128 + v * 16, _L)] = sl
            pltpu.async_copy(acc.at[slotblk.at[pl.ds(vb * 128, 128)]],
                             rows, sem).wait()
            pltpu.sync_copy(rows, out.at[cid, pl.ds(jbase + vb * 128, 128)])

    return sc_fn


def _tc_body(acc_ref, w_ref, b_ref, o_ref):
    a = acc_ref[0] + acc_ref[1]
    p = lax.dot_general(a, w_ref[...], (((1,), (1,)), ((), ())),
                        preferred_element_type=jnp.float32)
    p = p + b_ref[...]
    u = p[:_B]
    t = p[_B:]
    o_ref[...] = jnp.sum(u * t, axis=1, keepdims=True)


def kernel(user_indices, item_indices, edge_index, user_table, item_table,
           W, b):
    needed = jnp.concatenate([user_indices, item_indices + _NU])
    feat = jnp.concatenate([user_table, item_table], axis=0)
    zeros2d = jnp.zeros((_ZROWS, _D), jnp.float32)
    neg1 = jnp.full((_NTOT,), -1, jnp.int32)

    sc_fn = _make_sc_kernel()
    partials = sc_fn(edge_index[0], edge_index[1], needed, feat, zeros2d,
                     neg1)

    out = pl.pallas_call(
        _tc_body,
        out_shape=jax.ShapeDtypeStruct((_B, 1), jnp.float32),
    )(partials, W, jnp.reshape(b, (1, _D)))
    return out


# speedup vs baseline: 20.8703x; 1.1657x over previous
import functools

import jax
import jax.numpy as jnp
from jax import lax
from jax.experimental import pallas as pl
from jax.experimental.pallas import tpu as pltpu
from jax.experimental.pallas import tpu_sc as plsc

_NU = 25000
_NTOT = 50000
_D = 64
_NE = 800000
_B = 4096
_NSLOT = 2 * _B
_L = 16
_NS = 16
_NC = 2
_NW = _NC * _NS

_NVEC = _NE // _L
_VEC_LO = _NVEC // _NW
_VEC_HI = _VEC_LO + 1
_N_HI = _NVEC - _NW * _VEC_LO
_CHUNK_V = 128
_CHUNK_E = _CHUNK_V * _L
_N_CHUNKS = (_VEC_HI + _CHUNK_V - 1) // _CHUNK_V
_CMAX = ((_VEC_HI * _L + 127) // 128) * 128
_BLK = 64
_DUMMY = _NSLOT
_ACC_ROWS = _NSLOT + 128
_ZROWS = _ACC_ROWS // _NS
_JPT = _NSLOT // _NS
_NCH = 2048


def _make_sc_kernel():
    mesh = plsc.VectorSubcoreMesh(core_axis_name="c", subcore_axis_name="s")

    @functools.partial(
        pl.kernel,
        out_type=jax.ShapeDtypeStruct((_NC, _NSLOT, _D), jnp.float32),
        mesh=mesh,
        scratch_types=[
            pltpu.VMEM((_NTOT,), jnp.int32),
            pltpu.VMEM((_NCH,), jnp.int32),
            pltpu.VMEM((2 * _CHUNK_E,), jnp.int32),
            pltpu.VMEM((2 * _CHUNK_E,), jnp.int32),
            pltpu.VMEM((_CMAX,), jnp.int32),
            pltpu.VMEM((2 * _BLK,), jnp.int32),
            pltpu.VMEM((_BLK,), jnp.int32),
            pltpu.VMEM((2 * _BLK, _D), jnp.float32),
            pltpu.VMEM((_JPT,), jnp.int32),
            pltpu.VMEM((_L,), jnp.int32),
            pltpu.VMEM_SHARED((_ACC_ROWS, _D), jnp.float32),
            pltpu.SemaphoreType.DMA,
            pltpu.SemaphoreType.DMA,
        ],
        compiler_params=pltpu.CompilerParams(needs_layout_passes=False,
                                             use_tc_tiling_on_sc=False),
    )
    def sc_fn(esrc, edst, needed, feat, zeros2d, neg1, out,
              map_ref, nbuf, dstbuf, srcbuf, cpk, sstage, tstage, rows,
              slotblk, cnt_ref, acc, sem, sem2):
        cid = lax.axis_index("c")
        sid = lax.axis_index("s")
        wid = cid * _NS + sid

        iota = lax.iota(jnp.int32, _L)

        zbase = pl.multiple_of(sid * _ZROWS, 8)
        pltpu.sync_copy(zeros2d.at[pl.ds(0, _ZROWS)],
                        acc.at[pl.ds(zbase, _ZROWS)])

        pltpu.sync_copy(neg1, map_ref)
        for c in range(_NSLOT // _NCH):
            pltpu.sync_copy(needed.at[pl.ds(c * _NCH, _NCH)], nbuf)

            def _map_body(i, carry, _c=c):
                for u in range(4):
                    off = pl.multiple_of(i * 64 + u * 16, 16)
                    vals = nbuf[pl.ds(off, _L)]
                    plsc.store_scatter(map_ref, [vals],
                                       iota + off + _c * _NCH)
                return carry

            lax.fori_loop(0, _NCH // 64, _map_body, 0)

        plsc.subcore_barrier()

        base_vec = jnp.where(wid < _N_HI, wid * _VEC_HI,
                             _N_HI * _VEC_HI + (wid - _N_HI) * _VEC_LO)
        n_vec = jnp.where(wid < _N_HI, _VEC_HI, _VEC_LO)

        def _chunk_dma_base(k):
            chunk_lo = base_vec * _L + k * _CHUNK_E
            return pl.multiple_of(jnp.minimum(chunk_lo, _NE - _CHUNK_E), 8)

        def _fire_chunk(k):
            p = (k % 2) * _CHUNK_E
            dmab = _chunk_dma_base(k)
            pltpu.async_copy(edst.at[pl.ds(dmab, _CHUNK_E)],
                             dstbuf.at[pl.ds(p, _CHUNK_E)], sem2)
            pltpu.async_copy(esrc.at[pl.ds(dmab, _CHUNK_E)],
                             srcbuf.at[pl.ds(p, _CHUNK_E)], sem2)

        def _wait_chunk():
            pltpu.make_async_copy(edst.at[pl.ds(0, _CHUNK_E)],
                                  dstbuf.at[pl.ds(0, _CHUNK_E)], sem2).wait()
            pltpu.make_async_copy(edst.at[pl.ds(0, _CHUNK_E)],
                                  srcbuf.at[pl.ds(0, _CHUNK_E)], sem2).wait()

        def _filter_vec(voff, cnt, pbase):
            off = pl.multiple_of(voff * _L + pbase, 16)
            d = dstbuf[pl.ds(off, _L)]
            s = srcbuf[pl.ds(off, _L)]
            slot = plsc.load_gather(map_ref, [d])
            m = slot >= 0
            pos = cnt + plsc.cumsum(m.astype(jnp.int32)) - 1
            packed = lax.bitwise_or(s, lax.shift_left(slot, 17))
            plsc.store_scatter(cpk, [pos], packed, mask=m)
            return cnt + plsc.all_reduce_population_count(m)

        cnt = jnp.zeros((_L,), jnp.int32)
        _fire_chunk(0)
        for k in range(_N_CHUNKS):
            pbase = (k % 2) * _CHUNK_E
            if k + 1 < _N_CHUNKS:
                _fire_chunk(k + 1)
            _wait_chunk()
            if k < _N_CHUNKS - 1:
                def _quad(i, c, _pb=pbase):
                    for u in range(4):
                        c = _filter_vec(i * 4 + u, c, _pb)
                    return c
                cnt = lax.fori_loop(0, _CHUNK_V // 4, _quad, cnt)
            else:
                off_vec = lax.shift_right_logical(
                    base_vec * _L + k * _CHUNK_E - _chunk_dma_base(k), 4)
                nv = jnp.clip(n_vec - k * _CHUNK_V, 0, _CHUNK_V)

                def _one(i, c, _pb=pbase, _ov=off_vec):
                    return _filter_vec(_ov + i, c, _pb)
                cnt = lax.fori_loop(0, nv, _one, cnt)

        cnt_ref[...] = cnt
        n = cnt_ref[...][0]
        n_pad = lax.bitwise_and(n + _BLK - 1, ~(_BLK - 1))

        def _pad_vec(v, carry):
            pos = iota + v * _L
            mpad = pos >= n
            plsc.store_scatter(cpk, [pos],
                               jnp.full((_L,), _DUMMY << 17, jnp.int32),
                               mask=mpad)
            return carry

        lax.fori_loop(lax.shift_right_logical(n, 4),
                      lax.shift_right_logical(n_pad, 4), _pad_vec, 0)

        nb = lax.shift_right_logical(n_pad, 6)

        def _fire_block(j, half):
            hbase = pl.multiple_of(half * _BLK, 8)
            for v in range(4):
                off = pl.multiple_of(j * _BLK + v * 16, 16)
                w = cpk[pl.ds(off, _L)]
                sstage[pl.ds(pl.multiple_of(hbase + v * 16, 16), _L)] = \
                    lax.bitwise_and(w, (1 << 17) - 1)
            pltpu.async_copy(feat.at[sstage.at[pl.ds(hbase, _BLK)]],
                             rows.at[pl.ds(hbase, _BLK)], sem)

        @pl.when(nb > 0)
        def _():
            _fire_block(0, jnp.int32(0))

        def _blk_body(j, carry):
            p = lax.bitwise_and(j, 1)

            @pl.when(j + 1 < nb)
            def _():
                _fire_block(j + 1, 1 - p)

            pltpu.make_async_copy(feat.at[sstage.at[pl.ds(0, _BLK)]],
                                  rows.at[pl.ds(0, _BLK)], sem).wait()
            for v in range(4):
                off = pl.multiple_of(j * _BLK + v * 16, 16)
                w = cpk[pl.ds(off, _L)]
                tstage[pl.ds(v * 16, _L)] = lax.shift_right_logical(w, 17)
            pbase = pl.multiple_of(p * _BLK, 8)
            pltpu.sync_copy(rows.at[pl.ds(pbase, _BLK)], acc.at[tstage],
                            add=True)
            return carry

        lax.fori_loop(0, nb, _blk_body, 0)

        plsc.subcore_barrier()

        jbase = pl.multiple_of(sid * _JPT, 16)
        pltpu.sync_copy(needed.at[pl.ds(jbase, _JPT)],
                        nbuf.at[pl.ds(0, _JPT)])
        for vb in range(_JPT // 128):
            for v in range(8):
                off = pl.multiple_of(vb * 128 + v * 16, 16)
                vals = nbuf[pl.ds(off, _L)]
                sl = plsc.load_gather(map_ref, [vals])
                slotblk[pl.ds(off, _L)] = sl
            pltpu.async_copy(acc.at[slotblk.at[pl.ds(vb * 128, 128)]],
                             rows.at[pl.ds(0, 128)], sem).wait()
            pltpu.sync_copy(rows.at[pl.ds(0, 128)],
                            out.at[cid, pl.ds(jbase + vb * 128, 128)])

    return sc_fn


def _tc_body(acc_ref, w_ref, b_ref, o_ref):
    a = acc_ref[0] + acc_ref[1]
    p = lax.dot_general(a, w_ref[...], (((1,), (1,)), ((), ())),
                        preferred_element_type=jnp.float32)
    p = p + b_ref[...]
    u = p[:_B]
    t = p[_B:]
    o_ref[...] = jnp.sum(u * t, axis=1, keepdims=True)


def kernel(user_indices, item_indices, edge_index, user_table, item_table,
           W, b):
    needed = jnp.concatenate([user_indices, item_indices + _NU])
    feat = jnp.concatenate([user_table, item_table], axis=0)
    zeros2d = jnp.zeros((_ZROWS, _D), jnp.float32)
    neg1 = jnp.full((_NTOT,), -1, jnp.int32)

    sc_fn = _make_sc_kernel()
    partials = sc_fn(edge_index[0], edge_index[1], needed, feat, zeros2d,
                     neg1)

    out = pl.pallas_call(
        _tc_body,
        out_shape=jax.ShapeDtypeStruct((_B, 1), jnp.float32),
    )(partials, W, jnp.reshape(b, (1, _D)))
    return out


# speedup vs baseline: 23.6623x; 1.1338x over previous
import functools

import jax
import jax.numpy as jnp
from jax import lax
from jax.experimental import pallas as pl
from jax.experimental.pallas import tpu as pltpu
from jax.experimental.pallas import tpu_sc as plsc

_NU = 25000
_NTOT = 50000
_D = 64
_NE = 800000
_B = 4096
_NSLOT = 2 * _B
_L = 16
_NS = 16
_NC = 2
_NW = _NC * _NS

_NT128 = _NE // 128
_T_LO = _NT128 // _NW
_T_HI = _T_LO + 1
_N_HI = _NT128 - _NW * _T_LO
_VEC_HI = _T_HI * 8
_VEC_LO = _T_LO * 8
_CHUNK_V = 128
_CHUNK_E = _CHUNK_V * _L
_N_CHUNKS = (_VEC_HI + _CHUNK_V - 1) // _CHUNK_V
_CMAX = _T_HI * 128
_BLK = 128
_NQ = 4
_DUMMY = _NSLOT
_ACC_ROWS = _NSLOT + 128
_ZROWS = _ACC_ROWS // _NS
_JPT = _NSLOT // _NS


def _make_filter_kernel():
    mesh = plsc.VectorSubcoreMesh(core_axis_name="c", subcore_axis_name="s")

    @functools.partial(
        pl.kernel,
        out_type=(
            jax.ShapeDtypeStruct((_NW * _CMAX,), jnp.int32),
            jax.ShapeDtypeStruct((_NW * _L,), jnp.int32),
            jax.ShapeDtypeStruct((_NSLOT,), jnp.int32),
        ),
        mesh=mesh,
        scratch_types=[
            pltpu.VMEM((_NTOT,), jnp.int32),
            pltpu.VMEM((_NSLOT,), jnp.int32),
            pltpu.VMEM((2, _CHUNK_E), jnp.int32),
            pltpu.VMEM((2, _CHUNK_E), jnp.int32),
            pltpu.VMEM((_CMAX,), jnp.int32),
            pltpu.VMEM((_NSLOT // _NW,), jnp.int32),
            pltpu.VMEM((_L,), jnp.int32),
            pltpu.SemaphoreType.DMA,
        ],
        compiler_params=pltpu.CompilerParams(needs_layout_passes=False,
                                             use_tc_tiling_on_sc=False),
    )
    def f_fn(edges, needed, neg1, cpkd, meta, sfixg,
             map_ref, nbuf, ebufa, ebufb, cpk, sfix, cnt_ref, sem2):
        cid = lax.axis_index("c")
        sid = lax.axis_index("s")
        wid = cid * _NS + sid
        iota = lax.iota(jnp.int32, _L)

        pltpu.sync_copy(needed, nbuf)
        pltpu.sync_copy(neg1, map_ref)

        def _map_body(i, carry):
            for u in range(4):
                off = pl.multiple_of(i * 64 + u * 16, 16)
                vals = nbuf[pl.ds(off, _L)]
                plsc.store_scatter(map_ref, [vals], iota + off)
            return carry

        lax.fori_loop(0, _NSLOT // 64, _map_body, 0)

        jb = pl.multiple_of(wid * (_NSLOT // _NW), 16)
        for v in range(_NSLOT // _NW // _L):
            vals = nbuf[pl.ds(pl.multiple_of(jb + v * 16, 16), _L)]
            sfix[pl.ds(v * 16, _L)] = plsc.load_gather(map_ref, [vals])
        pltpu.sync_copy(sfix, sfixg.at[pl.ds(jb, _NSLOT // _NW)])

        base_e = jnp.where(wid < _N_HI, wid * _T_HI * 128,
                           _N_HI * _T_HI * 128 + (wid - _N_HI) * _T_LO * 128)
        n_vec = jnp.where(wid < _N_HI, _VEC_HI, _VEC_LO)

        def _chunk_dma_base(k):
            chunk_lo = base_e + k * _CHUNK_E
            return pl.multiple_of(jnp.minimum(chunk_lo, _NE - _CHUNK_E), 128)

        def _fire_chunk(k):
            buf = ebufa if k % 2 == 0 else ebufb
            pltpu.async_copy(edges.at[:, pl.ds(_chunk_dma_base(k), _CHUNK_E)],
                             buf, sem2)

        def _wait_chunk():
            pltpu.make_async_copy(edges.at[:, pl.ds(0, _CHUNK_E)],
                                  ebufa, sem2).wait()

        def _filter_vec(voff, cnts, buf):
            cnt_u, cnt_i = cnts
            off = pl.multiple_of(voff * _L, 16)
            d = buf[1, pl.ds(off, _L)]
            s = buf[0, pl.ds(off, _L)]
            slot = plsc.load_gather(map_ref, [d])
            m = slot >= 0
            is_u = s < _NU
            m_u = jnp.logical_and(m, is_u)
            m_i = jnp.logical_and(m, jnp.logical_not(is_u))
            sh = lax.shift_left(slot, 17)
            pos_u = cnt_u + plsc.cumsum(m_u.astype(jnp.int32)) - 1
            plsc.store_scatter(cpk, [pos_u], lax.bitwise_or(s, sh), mask=m_u)
            pos_i = (_CMAX - 1) - (cnt_i + plsc.cumsum(m_i.astype(jnp.int32))
                                   - 1)
            plsc.store_scatter(cpk, [pos_i],
                               lax.bitwise_or(s - _NU, sh), mask=m_i)
            return (cnt_u + plsc.all_reduce_population_count(m_u),
                    cnt_i + plsc.all_reduce_population_count(m_i))

        cnt = (jnp.zeros((_L,), jnp.int32), jnp.zeros((_L,), jnp.int32))
        _fire_chunk(0)
        for k in range(_N_CHUNKS):
            buf = ebufa if k % 2 == 0 else ebufb
            if k + 1 < _N_CHUNKS:
                _fire_chunk(k + 1)
            _wait_chunk()
            if k < _N_CHUNKS - 1:
                def _quad(i, c, _b=buf):
                    for u in range(4):
                        c = _filter_vec(i * 4 + u, c, _b)
                    return c
                cnt = lax.fori_loop(0, _CHUNK_V // 4, _quad, cnt)
            else:
                off_vec = lax.shift_right_logical(
                    base_e + k * _CHUNK_E - _chunk_dma_base(k), 4)
                nv = jnp.clip(n_vec - k * _CHUNK_V, 0, _CHUNK_V)

                def _one(i, c, _b=buf, _ov=off_vec):
                    return _filter_vec(_ov + i, c, _b)
                cnt = lax.fori_loop(0, nv, _one, cnt)

        cnt_ref[...] = cnt[0]
        n_u = cnt_ref[...][0]
        pltpu.sync_copy(cnt_ref.at[pl.ds(0, 8)],
                        meta.at[pl.ds(pl.multiple_of(wid * _L, 8), 8)])
        cnt_ref[...] = cnt[1]
        n_i = cnt_ref[...][0]
        pltpu.sync_copy(cnt_ref.at[pl.ds(0, 8)],
                        meta.at[pl.ds(pl.multiple_of(wid * _L + 8, 8), 8)])
        npad_u = lax.bitwise_and(n_u + _BLK - 1, ~(_BLK - 1))
        npad_i = lax.bitwise_and(n_i + _BLK - 1, ~(_BLK - 1))
        dummy = jnp.full((_L,), _DUMMY << 17, jnp.int32)

        def _pad_u(v, carry):
            pos = iota + v * _L
            plsc.store_scatter(cpk, [pos], dummy, mask=pos >= n_u)
            return carry

        lax.fori_loop(lax.shift_right_logical(n_u, 4),
                      lax.shift_right_logical(npad_u, 4), _pad_u, 0)

        def _pad_i(v, carry):
            pos = iota + v * _L
            plsc.store_scatter(cpk, [pos], dummy, mask=pos < _CMAX - n_i)
            return carry

        lax.fori_loop(lax.shift_right_logical(_CMAX - npad_i, 4),
                      lax.shift_right_logical(_CMAX - n_i, 4)
                      + jnp.where(lax.bitwise_and(_CMAX - n_i, 15) > 0, 1, 0),
                      _pad_i, 0)

        pltpu.sync_copy(cpk, cpkd.at[pl.ds(wid * _CMAX, _CMAX)])

    return f_fn


def _make_agg_kernel():
    mesh = plsc.VectorSubcoreMesh(core_axis_name="c", subcore_axis_name="s")

    @functools.partial(
        pl.kernel,
        out_type=jax.ShapeDtypeStruct((_NC, _NSLOT, _D), jnp.float32),
        mesh=mesh,
        scratch_types=[
            pltpu.VMEM((_CMAX,), jnp.int32),
            pltpu.VMEM((_L,), jnp.int32),
            pltpu.VMEM((_NQ * _BLK,), jnp.int32),
            pltpu.VMEM((_BLK,), jnp.int32),
            pltpu.VMEM((_NQ * _BLK, _D), jnp.float32),
            pltpu.VMEM((_JPT,), jnp.int32),
            pltpu.VMEM_SHARED((_ACC_ROWS, _D), jnp.float32),
            pltpu.SemaphoreType.DMA,
        ],
        compiler_params=pltpu.CompilerParams(needs_layout_passes=False,
                                             use_tc_tiling_on_sc=False),
    )
    def a_fn(cpkd, meta, sfixg, utab, itab, zeros2d, out,
             cpk, metab, sstage, tstage, rows, sfixb, acc, sem):
        cid = lax.axis_index("c")
        sid = lax.axis_index("s")
        wid = cid * _NS + sid

        pltpu.sync_copy(zeros2d.at[pl.ds(0, _ZROWS)],
                        acc.at[pl.ds(pl.multiple_of(sid * _ZROWS, 8), _ZROWS)])
        pltpu.sync_copy(cpkd.at[pl.ds(pl.multiple_of(wid * _CMAX, 8), _CMAX)],
                        cpk)
        pltpu.sync_copy(meta.at[pl.ds(pl.multiple_of(wid * _L, 8), _L)],
                        metab)
        mv = metab[...]
        n_u = mv[0]
        n_i = mv[8]
        npad_u = lax.bitwise_and(n_u + _BLK - 1, ~(_BLK - 1))
        npad_i = lax.bitwise_and(n_i + _BLK - 1, ~(_BLK - 1))
        nb_u = lax.shift_right_logical(npad_u, 7)
        nb = nb_u + lax.shift_right_logical(npad_i, 7)
        ibase = _CMAX - npad_i

        def _blk_off(j):
            return jnp.where(j < nb_u, j * _BLK,
                             ibase + (j - nb_u) * _BLK)

        plsc.subcore_barrier()

        def _fire_block(j, half):
            hbase = pl.multiple_of(half * _BLK, 8)
            boff = _blk_off(j)
            for v in range(_BLK // _L):
                w = cpk[pl.ds(pl.multiple_of(boff + v * 16, 16), _L)]
                sstage[pl.ds(pl.multiple_of(hbase + v * 16, 16), _L)] = \
                    lax.bitwise_and(w, (1 << 17) - 1)

            @pl.when(j < nb_u)
            def _():
                pltpu.async_copy(utab.at[sstage.at[pl.ds(hbase, _BLK)]],
                                 rows.at[pl.ds(hbase, _BLK)], sem)

            @pl.when(j >= nb_u)
            def _():
                pltpu.async_copy(itab.at[sstage.at[pl.ds(hbase, _BLK)]],
                                 rows.at[pl.ds(hbase, _BLK)], sem)

        for q in range(_NQ - 1):
            @pl.when(q < nb)
            def _(_q=q):
                _fire_block(jnp.int32(_q), jnp.int32(_q))

        def _blk_body(j, carry):
            @pl.when(j + (_NQ - 1) < nb)
            def _():
                _fire_block(j + (_NQ - 1),
                            lax.bitwise_and(j + (_NQ - 1), _NQ - 1))

            pltpu.make_async_copy(utab.at[sstage.at[pl.ds(0, _BLK)]],
                                  rows.at[pl.ds(0, _BLK)], sem).wait()
            boff = _blk_off(j)
            for v in range(_BLK // _L):
                w = cpk[pl.ds(pl.multiple_of(boff + v * 16, 16), _L)]
                tstage[pl.ds(v * 16, _L)] = lax.shift_right_logical(w, 17)
            pbase = pl.multiple_of(lax.bitwise_and(j, _NQ - 1) * _BLK, 8)
            pltpu.sync_copy(rows.at[pl.ds(pbase, _BLK)], acc.at[tstage],
                            add=True)
            return carry

        lax.fori_loop(0, nb, _blk_body, 0)

        plsc.subcore_barrier()

        jbase = pl.multiple_of(sid * _JPT, 16)
        pltpu.sync_copy(sfixg.at[pl.ds(jbase, _JPT)], sfixb)
        for vb in range(_JPT // _BLK):
            pltpu.async_copy(acc.at[sfixb.at[pl.ds(vb * _BLK, _BLK)]],
                             rows.at[pl.ds(0, _BLK)], sem).wait()
            pltpu.sync_copy(rows.at[pl.ds(0, _BLK)],
                            out.at[cid, pl.ds(jbase + vb * _BLK, _BLK)])

    return a_fn


def _tc_body(acc_ref, w_ref, b_ref, o_ref):
    a = acc_ref[0] + acc_ref[1]
    p = lax.dot_general(a, w_ref[...], (((1,), (1,)), ((), ())),
                        preferred_element_type=jnp.float32)
    p = p + b_ref[...]
    u = p[:_B]
    t = p[_B:]
    o_ref[...] = jnp.sum(u * t, axis=1, keepdims=True)


def kernel(user_indices, item_indices, edge_index, user_table, item_table,
           W, b):
    needed = jnp.concatenate([user_indices, item_indices + _NU])
    zeros2d = jnp.zeros((_ZROWS, _D), jnp.float32)
    neg1 = jnp.full((_NTOT,), -1, jnp.int32)

    cpkd, meta, sfixg = _make_filter_kernel()(edge_index, needed, neg1)
    partials = _make_agg_kernel()(cpkd, meta, sfixg, user_table, item_table,
                                  zeros2d)

    out = pl.pallas_call(
        _tc_body,
        out_shape=jax.ShapeDtypeStruct((_B, 1), jnp.float32),
    )(partials, W, jnp.reshape(b, (1, _D)))
    return out


# speedup vs baseline: 23.8057x; 1.0061x over previous
import functools

import jax
import jax.numpy as jnp
from jax import lax
from jax.experimental import pallas as pl
from jax.experimental.pallas import tpu as pltpu
from jax.experimental.pallas import tpu_sc as plsc

_NU = 25000
_NTOT = 50000
_D = 64
_NE = 800000
_B = 4096
_NSLOT = 2 * _B
_L = 16
_NS = 16
_NC = 2
_NW = _NC * _NS

_NT128 = _NE // 128
_T_LO = _NT128 // _NW
_T_HI = _T_LO + 1
_N_HI = _NT128 - _NW * _T_LO
_VEC_HI = _T_HI * 8
_VEC_LO = _T_LO * 8
_CHUNK_V = 128
_CHUNK_E = _CHUNK_V * _L
_N_CHUNKS = (_VEC_HI + _CHUNK_V - 1) // _CHUNK_V
_CMAX = _T_HI * 128
_BLK = 128
_NQ = 4
_DUMMY = _NSLOT
_ACC_ROWS = _NSLOT + 128
_ZROWS = _ACC_ROWS // _NS
_JPT = _NSLOT // _NS


def _make_filter_kernel():
    mesh = plsc.VectorSubcoreMesh(core_axis_name="c", subcore_axis_name="s")

    @functools.partial(
        pl.kernel,
        out_type=(
            jax.ShapeDtypeStruct((_NW * _CMAX,), jnp.int32),
            jax.ShapeDtypeStruct((_NW * _L,), jnp.int32),
            jax.ShapeDtypeStruct((_NSLOT,), jnp.int32),
        ),
        mesh=mesh,
        scratch_types=[
            pltpu.VMEM((_NTOT,), jnp.int32),
            pltpu.VMEM((_NSLOT,), jnp.int32),
            pltpu.VMEM((2, _CHUNK_E), jnp.int32),
            pltpu.VMEM((2, _CHUNK_E), jnp.int32),
            pltpu.VMEM((_CMAX,), jnp.int32),
            pltpu.VMEM((_NSLOT // _NW,), jnp.int32),
            pltpu.VMEM((_L,), jnp.int32),
            pltpu.SemaphoreType.DMA,
        ],
        compiler_params=pltpu.CompilerParams(needs_layout_passes=False,
                                             use_tc_tiling_on_sc=False),
    )
    def f_fn(edges, needed, neg1, cpkd, meta, sfixg,
             map_ref, nbuf, ebufa, ebufb, cpk, sfix, cnt_ref, sem2):
        cid = lax.axis_index("c")
        sid = lax.axis_index("s")
        wid = cid * _NS + sid
        iota = lax.iota(jnp.int32, _L)

        pltpu.sync_copy(needed, nbuf)
        pltpu.sync_copy(neg1, map_ref)

        def _map_body(i, carry):
            for u in range(4):
                off = pl.multiple_of(i * 64 + u * 16, 16)
                vals = nbuf[pl.ds(off, _L)]
                plsc.store_scatter(map_ref, [vals], iota + off)
            return carry

        lax.fori_loop(0, _NSLOT // 64, _map_body, 0)

        jb = pl.multiple_of(wid * (_NSLOT // _NW), 16)
        for v in range(_NSLOT // _NW // _L):
            vals = nbuf[pl.ds(pl.multiple_of(jb + v * 16, 16), _L)]
            sfix[pl.ds(v * 16, _L)] = plsc.load_gather(map_ref, [vals])
        pltpu.sync_copy(sfix, sfixg.at[pl.ds(jb, _NSLOT // _NW)])

        base_e = jnp.where(wid < _N_HI, wid * _T_HI * 128,
                           _N_HI * _T_HI * 128 + (wid - _N_HI) * _T_LO * 128)
        n_vec = jnp.where(wid < _N_HI, _VEC_HI, _VEC_LO)

        def _chunk_dma_base(k):
            chunk_lo = base_e + k * _CHUNK_E
            return pl.multiple_of(jnp.minimum(chunk_lo, _NE - _CHUNK_E), 128)

        def _fire_chunk(k):
            buf = ebufa if k % 2 == 0 else ebufb
            pltpu.async_copy(edges.at[:, pl.ds(_chunk_dma_base(k), _CHUNK_E)],
                             buf, sem2)

        def _wait_chunk():
            pltpu.make_async_copy(edges.at[:, pl.ds(0, _CHUNK_E)],
                                  ebufa, sem2).wait()

        def _filter_vec(voff, cnts, buf):
            cnt_u, cnt_i = cnts
            off = pl.multiple_of(voff * _L, 16)
            d = buf[1, pl.ds(off, _L)]
            s = buf[0, pl.ds(off, _L)]
            slot = plsc.load_gather(map_ref, [d])
            m = slot >= 0
            is_u = s < _NU
            m_u = jnp.logical_and(m, is_u)
            m_i = jnp.logical_and(m, jnp.logical_not(is_u))
            sh = lax.shift_left(slot, 17)
            pos_u = cnt_u + plsc.cumsum(m_u.astype(jnp.int32)) - 1
            plsc.store_scatter(cpk, [pos_u], lax.bitwise_or(s, sh), mask=m_u)
            pos_i = (_CMAX - 1) - (cnt_i + plsc.cumsum(m_i.astype(jnp.int32))
                                   - 1)
            plsc.store_scatter(cpk, [pos_i],
                               lax.bitwise_or(s - _NU, sh), mask=m_i)
            return (cnt_u + plsc.all_reduce_population_count(m_u),
                    cnt_i + plsc.all_reduce_population_count(m_i))

        cnt = (jnp.zeros((_L,), jnp.int32), jnp.zeros((_L,), jnp.int32))
        _fire_chunk(0)
        for k in range(_N_CHUNKS):
            buf = ebufa if k % 2 == 0 else ebufb
            if k + 1 < _N_CHUNKS:
                _fire_chunk(k + 1)
            _wait_chunk()
            if k < _N_CHUNKS - 1:
                def _quad(i, c, _b=buf):
                    for u in range(4):
                        c = _filter_vec(i * 4 + u, c, _b)
                    return c
                cnt = lax.fori_loop(0, _CHUNK_V // 4, _quad, cnt)
            else:
                off_vec = lax.shift_right_logical(
                    base_e + k * _CHUNK_E - _chunk_dma_base(k), 4)
                nv = jnp.clip(n_vec - k * _CHUNK_V, 0, _CHUNK_V)

                def _one(i, c, _b=buf, _ov=off_vec):
                    return _filter_vec(_ov + i, c, _b)
                cnt = lax.fori_loop(0, nv, _one, cnt)

        cnt_ref[...] = cnt[0]
        n_u = cnt_ref[...][0]
        pltpu.sync_copy(cnt_ref.at[pl.ds(0, 8)],
                        meta.at[pl.ds(pl.multiple_of(wid * _L, 8), 8)])
        cnt_ref[...] = cnt[1]
        n_i = cnt_ref[...][0]
        pltpu.sync_copy(cnt_ref.at[pl.ds(0, 8)],
                        meta.at[pl.ds(pl.multiple_of(wid * _L + 8, 8), 8)])
        npad_u = lax.bitwise_and(n_u + _BLK - 1, ~(_BLK - 1))
        npad_i = lax.bitwise_and(n_i + _BLK - 1, ~(_BLK - 1))
        dummy = jnp.full((_L,), _DUMMY << 17, jnp.int32)

        def _pad_u(v, carry):
            pos = iota + v * _L
            plsc.store_scatter(cpk, [pos], dummy, mask=pos >= n_u)
            return carry

        lax.fori_loop(lax.shift_right_logical(n_u, 4),
                      lax.shift_right_logical(npad_u, 4), _pad_u, 0)

        def _pad_i(v, carry):
            pos = iota + v * _L
            plsc.store_scatter(cpk, [pos], dummy, mask=pos < _CMAX - n_i)
            return carry

        lax.fori_loop(lax.shift_right_logical(_CMAX - npad_i, 4),
                      lax.shift_right_logical(_CMAX - n_i, 4)
                      + jnp.where(lax.bitwise_and(_CMAX - n_i, 15) > 0, 1, 0),
                      _pad_i, 0)

        pltpu.sync_copy(cpk, cpkd.at[pl.ds(wid * _CMAX, _CMAX)])

    return f_fn


def _make_agg_kernel():
    mesh = plsc.VectorSubcoreMesh(core_axis_name="c", subcore_axis_name="s")

    @functools.partial(
        pl.kernel,
        out_type=jax.ShapeDtypeStruct((_NC, _NSLOT, _D), jnp.float32),
        mesh=mesh,
        scratch_types=[
            pltpu.VMEM((_CMAX,), jnp.int32),
            pltpu.VMEM((_L,), jnp.int32),
            pltpu.VMEM((_NQ * _BLK,), jnp.int32),
            pltpu.VMEM((_BLK,), jnp.int32),
            pltpu.VMEM((_BLK,), jnp.int32),
            pltpu.VMEM((_BLK,), jnp.int32),
            pltpu.VMEM((_BLK,), jnp.int32),
            pltpu.VMEM((_NQ * _BLK, _D), jnp.float32),
            pltpu.VMEM((_JPT,), jnp.int32),
            pltpu.VMEM_SHARED((_ACC_ROWS, _D), jnp.float32),
            pltpu.SemaphoreType.DMA,
            pltpu.SemaphoreType.DMA,
        ],
        compiler_params=pltpu.CompilerParams(needs_layout_passes=False,
                                             use_tc_tiling_on_sc=False),
    )
    def a_fn(cpkd, meta, sfixg, utab, itab, zeros2d, out,
             cpk, metab, sstage, tst0, tst1, tst2, tst3, rows, sfixb, acc,
             sem, sem3):
        cid = lax.axis_index("c")
        sid = lax.axis_index("s")
        wid = cid * _NS + sid

        pltpu.sync_copy(zeros2d.at[pl.ds(0, _ZROWS)],
                        acc.at[pl.ds(pl.multiple_of(sid * _ZROWS, 8), _ZROWS)])
        pltpu.sync_copy(cpkd.at[pl.ds(pl.multiple_of(wid * _CMAX, 8), _CMAX)],
                        cpk)
        pltpu.sync_copy(meta.at[pl.ds(pl.multiple_of(wid * _L, 8), _L)],
                        metab)
        mv = metab[...]
        n_u = mv[0]
        n_i = mv[8]
        npad_u = lax.bitwise_and(n_u + _BLK - 1, ~(_BLK - 1))
        npad_i = lax.bitwise_and(n_i + _BLK - 1, ~(_BLK - 1))
        nb_u = lax.shift_right_logical(npad_u, 7)
        nb = nb_u + lax.shift_right_logical(npad_i, 7)
        ibase = _CMAX - npad_i

        def _blk_off(j):
            return jnp.where(j < nb_u, j * _BLK,
                             ibase + (j - nb_u) * _BLK)

        plsc.subcore_barrier()

        def _fire_block(j, half):
            hbase = pl.multiple_of(half * _BLK, 8)
            boff = _blk_off(j)
            for v in range(_BLK // _L):
                w = cpk[pl.ds(pl.multiple_of(boff + v * 16, 16), _L)]
                sstage[pl.ds(pl.multiple_of(hbase + v * 16, 16), _L)] = \
                    lax.bitwise_and(w, (1 << 17) - 1)

            @pl.when(j < nb_u)
            def _():
                pltpu.async_copy(utab.at[sstage.at[pl.ds(hbase, _BLK)]],
                                 rows.at[pl.ds(hbase, _BLK)], sem)

            @pl.when(j >= nb_u)
            def _():
                pltpu.async_copy(itab.at[sstage.at[pl.ds(hbase, _BLK)]],
                                 rows.at[pl.ds(hbase, _BLK)], sem)

        for q in range(_NQ - 1):
            @pl.when(q < nb)
            def _(_q=q):
                _fire_block(jnp.int32(_q), jnp.int32(_q))

        tsts = [tst0, tst1, tst2, tst3]

        def _wait_scatter():
            pltpu.make_async_copy(rows.at[pl.ds(0, _BLK)],
                                  acc.at[tst0], sem3).wait()

        def _blk_body(j, carry):
            @pl.when(j >= 1)
            def _():
                _wait_scatter()

            @pl.when(j + (_NQ - 1) < nb)
            def _():
                _fire_block(j + (_NQ - 1),
                            lax.bitwise_and(j + (_NQ - 1), _NQ - 1))

            pltpu.make_async_copy(utab.at[sstage.at[pl.ds(0, _BLK)]],
                                  rows.at[pl.ds(0, _BLK)], sem).wait()
            half = lax.bitwise_and(j, _NQ - 1)
            boff = _blk_off(j)
            for q in range(_NQ):
                @pl.when(half == q)
                def _(_q=q):
                    for v in range(_BLK // _L):
                        w = cpk[pl.ds(pl.multiple_of(boff + v * 16, 16), _L)]
                        tsts[_q][pl.ds(v * 16, _L)] = \
                            lax.shift_right_logical(w, 17)
                    pltpu.async_copy(rows.at[pl.ds(_q * _BLK, _BLK)],
                                     acc.at[tsts[_q]], sem3, add=True)
            return carry

        lax.fori_loop(0, nb, _blk_body, 0)

        @pl.when(nb > 0)
        def _():
            _wait_scatter()

        plsc.subcore_barrier()

        jbase = pl.multiple_of(sid * _JPT, 16)
        pltpu.sync_copy(sfixg.at[pl.ds(jbase, _JPT)], sfixb)
        for vb in range(_JPT // _BLK):
            pltpu.async_copy(acc.at[sfixb.at[pl.ds(vb * _BLK, _BLK)]],
                             rows.at[pl.ds(vb * _BLK, _BLK)], sem)
        for vb in range(_JPT // _BLK):
            pltpu.make_async_copy(acc.at[sfixb.at[pl.ds(0, _BLK)]],
                                  rows.at[pl.ds(0, _BLK)], sem).wait()
            pltpu.sync_copy(rows.at[pl.ds(vb * _BLK, _BLK)],
                            out.at[cid, pl.ds(jbase + vb * _BLK, _BLK)])

    return a_fn


def _tc_body(acc_ref, w_ref, b_ref, o_ref):
    a = acc_ref[0] + acc_ref[1]
    p = lax.dot_general(a, w_ref[...], (((1,), (1,)), ((), ())),
                        preferred_element_type=jnp.float32)
    p = p + b_ref[...]
    u = p[:_B]
    t = p[_B:]
    o_ref[...] = jnp.sum(u * t, axis=1, keepdims=True)


def kernel(user_indices, item_indices, edge_index, user_table, item_table,
           W, b):
    needed = jnp.concatenate([user_indices, item_indices + _NU])
    zeros2d = jnp.zeros((_ZROWS, _D), jnp.float32)
    neg1 = jnp.full((_NTOT,), -1, jnp.int32)

    cpkd, meta, sfixg = _make_filter_kernel()(edge_index, needed, neg1)
    partials = _make_agg_kernel()(cpkd, meta, sfixg, user_table, item_table,
                                  zeros2d)

    out = pl.pallas_call(
        _tc_body,
        out_shape=jax.ShapeDtypeStruct((_B, 1), jnp.float32),
    )(partials, W, jnp.reshape(b, (1, _D)))
    return out


# speedup vs baseline: 24.8509x; 1.0439x over previous
import functools

import jax
import jax.numpy as jnp
from jax import lax
from jax.experimental import pallas as pl
from jax.experimental.pallas import tpu as pltpu
from jax.experimental.pallas import tpu_sc as plsc

_NU = 25000
_NTOT = 50000
_D = 64
_NE = 800000
_B = 4096
_NSLOT = 2 * _B
_L = 16
_NS = 16
_NC = 2
_NW = _NC * _NS

_NT128 = _NE // 128
_T_LO = _NT128 // _NW
_T_HI = _T_LO + 1
_N_HI = _NT128 - _NW * _T_LO
_VEC_HI = _T_HI * 8
_VEC_LO = _T_LO * 8
_CHUNK_V = 128
_CHUNK_E = _CHUNK_V * _L
_N_CHUNKS = (_VEC_HI + _CHUNK_V - 1) // _CHUNK_V
_CMAX = ((_VEC_HI * _L + 63) // 64) * 64
_BLK = 64
_DUMMY = _NSLOT
_ACC_ROWS = _NSLOT + 128
_ZROWS = _ACC_ROWS // _NS
_JPT = _NSLOT // _NS
_NCH = 2048


def _make_sc_kernel():
    mesh = plsc.VectorSubcoreMesh(core_axis_name="c", subcore_axis_name="s")

    @functools.partial(
        pl.kernel,
        out_type=jax.ShapeDtypeStruct((_NC, _NSLOT, _D), jnp.float32),
        mesh=mesh,
        scratch_types=[
            pltpu.VMEM((_NTOT,), jnp.int32),
            pltpu.VMEM((_NCH,), jnp.int32),
            pltpu.VMEM((2, _CHUNK_E), jnp.int32),
            pltpu.VMEM((2, _CHUNK_E), jnp.int32),
            pltpu.VMEM((_CMAX,), jnp.int32),
            pltpu.VMEM((2 * _BLK,), jnp.int32),
            pltpu.VMEM((_BLK,), jnp.int32),
            pltpu.VMEM((_BLK,), jnp.int32),
            pltpu.VMEM((2 * _BLK, _D), jnp.float32),
            pltpu.VMEM((_JPT,), jnp.int32),
            pltpu.VMEM((_L,), jnp.int32),
            pltpu.VMEM_SHARED((_ACC_ROWS, _D), jnp.float32),
            pltpu.SemaphoreType.DMA,
            pltpu.SemaphoreType.DMA,
            pltpu.SemaphoreType.DMA,
        ],
        compiler_params=pltpu.CompilerParams(needs_layout_passes=False,
                                             use_tc_tiling_on_sc=False),
    )
    def sc_fn(edges, needed, utab, itab, zeros2d, neg1, out,
              map_ref, nbuf, ebufa, ebufb, cpk, sstage, tst0, tst1, rows,
              slotblk, cnt_ref, acc, sem, sem2, sem3):
        cid = lax.axis_index("c")
        sid = lax.axis_index("s")
        wid = cid * _NS + sid

        iota = lax.iota(jnp.int32, _L)

        zbase = pl.multiple_of(sid * _ZROWS, 8)
        pltpu.sync_copy(zeros2d.at[pl.ds(0, _ZROWS)],
                        acc.at[pl.ds(zbase, _ZROWS)])

        pltpu.sync_copy(neg1, map_ref)
        for c in range(_NSLOT // _NCH):
            pltpu.sync_copy(needed.at[pl.ds(c * _NCH, _NCH)], nbuf)

            def _map_body(i, carry, _c=c):
                for u in range(4):
                    off = pl.multiple_of(i * 64 + u * 16, 16)
                    vals = nbuf[pl.ds(off, _L)]
                    plsc.store_scatter(map_ref, [vals],
                                       iota + off + _c * _NCH)
                return carry

            lax.fori_loop(0, _NCH // 64, _map_body, 0)

        plsc.subcore_barrier()

        base_e = jnp.where(wid < _N_HI, wid * _T_HI * 128,
                           _N_HI * _T_HI * 128 + (wid - _N_HI) * _T_LO * 128)
        n_vec = jnp.where(wid < _N_HI, _VEC_HI, _VEC_LO)

        def _chunk_dma_base(k):
            chunk_lo = base_e + k * _CHUNK_E
            return pl.multiple_of(jnp.minimum(chunk_lo, _NE - _CHUNK_E), 128)

        def _fire_chunk(k):
            buf = ebufa if k % 2 == 0 else ebufb
            dmab = _chunk_dma_base(k)
            pltpu.async_copy(edges.at[:, pl.ds(dmab, _CHUNK_E)], buf, sem2)

        def _wait_chunk():
            pltpu.make_async_copy(edges.at[:, pl.ds(0, _CHUNK_E)],
                                  ebufa, sem2).wait()

        def _filter_vec(voff, cnts, buf):
            cnt_u, cnt_i = cnts
            off = pl.multiple_of(voff * _L, 16)
            d = buf[1, pl.ds(off, _L)]
            s = buf[0, pl.ds(off, _L)]
            slot = plsc.load_gather(map_ref, [d])
            m = slot >= 0
            is_u = s < _NU
            m_u = jnp.logical_and(m, is_u)
            m_i = jnp.logical_and(m, jnp.logical_not(is_u))
            sh = lax.shift_left(slot, 17)
            pos_u = cnt_u + plsc.cumsum(m_u.astype(jnp.int32)) - 1
            plsc.store_scatter(cpk, [pos_u], lax.bitwise_or(s, sh), mask=m_u)
            pos_i = (_CMAX - 1) - (cnt_i + plsc.cumsum(m_i.astype(jnp.int32))
                                   - 1)
            plsc.store_scatter(cpk, [pos_i],
                               lax.bitwise_or(s - _NU, sh), mask=m_i)
            return (cnt_u + plsc.all_reduce_population_count(m_u),
                    cnt_i + plsc.all_reduce_population_count(m_i))

        cnt = (jnp.zeros((_L,), jnp.int32), jnp.zeros((_L,), jnp.int32))
        _fire_chunk(0)
        for k in range(_N_CHUNKS):
            buf = ebufa if k % 2 == 0 else ebufb
            if k + 1 < _N_CHUNKS:
                _fire_chunk(k + 1)
            _wait_chunk()
            if k < _N_CHUNKS - 1:
                def _quad(i, c, _b=buf):
                    for u in range(4):
                        c = _filter_vec(i * 4 + u, c, _b)
                    return c
                cnt = lax.fori_loop(0, _CHUNK_V // 4, _quad, cnt)
            else:
                off_vec = lax.shift_right_logical(
                    base_e + k * _CHUNK_E - _chunk_dma_base(k), 4)
                nv = jnp.clip(n_vec - k * _CHUNK_V, 0, _CHUNK_V)

                def _one(i, c, _b=buf, _ov=off_vec):
                    return _filter_vec(_ov + i, c, _b)
                cnt = lax.fori_loop(0, nv, _one, cnt)

        cnt_ref[...] = cnt[0]
        n_u = cnt_ref[...][0]
        cnt_ref[...] = cnt[1]
        n_i = cnt_ref[...][0]
        npad_u = lax.bitwise_and(n_u + _BLK - 1, ~(_BLK - 1))
        npad_i = lax.bitwise_and(n_i + _BLK - 1, ~(_BLK - 1))
        dummy = jnp.full((_L,), _DUMMY << 17, jnp.int32)

        def _pad_u(v, carry):
            pos = iota + v * _L
            plsc.store_scatter(cpk, [pos], dummy, mask=pos >= n_u)
            return carry

        lax.fori_loop(lax.shift_right_logical(n_u, 4),
                      lax.shift_right_logical(npad_u, 4), _pad_u, 0)

        def _pad_i(v, carry):
            pos = iota + v * _L
            plsc.store_scatter(cpk, [pos], dummy, mask=pos < _CMAX - n_i)
            return carry

        lax.fori_loop(lax.shift_right_logical(_CMAX - npad_i, 4),
                      lax.shift_right_logical(_CMAX - n_i, 4)
                      + jnp.where(lax.bitwise_and(_CMAX - n_i, 15) > 0, 1, 0),
                      _pad_i, 0)

        nb_u = lax.shift_right_logical(npad_u, 6)
        nb = nb_u + lax.shift_right_logical(npad_i, 6)
        ibase = _CMAX - npad_i

        def _blk_off(j):
            return jnp.where(j < nb_u, j * _BLK,
                             ibase + (j - nb_u) * _BLK)

        def _fire_block(j, half):
            hbase = pl.multiple_of(half * _BLK, 8)
            boff = _blk_off(j)
            for v in range(4):
                off = pl.multiple_of(boff + v * 16, 16)
                w = cpk[pl.ds(off, _L)]
                sstage[pl.ds(pl.multiple_of(hbase + v * 16, 16), _L)] = \
                    lax.bitwise_and(w, (1 << 17) - 1)

            @pl.when(j < nb_u)
            def _():
                pltpu.async_copy(utab.at[sstage.at[pl.ds(hbase, _BLK)]],
                                 rows.at[pl.ds(hbase, _BLK)], sem)

            @pl.when(j >= nb_u)
            def _():
                pltpu.async_copy(itab.at[sstage.at[pl.ds(hbase, _BLK)]],
                                 rows.at[pl.ds(hbase, _BLK)], sem)

        @pl.when(nb > 0)
        def _():
            _fire_block(0, jnp.int32(0))

        tsts = [tst0, tst1]

        def _wait_scatter():
            pltpu.make_async_copy(rows.at[pl.ds(0, _BLK)],
                                  acc.at[tst0], sem3).wait()

        def _blk_body(j, carry):
            p = lax.bitwise_and(j, 1)

            @pl.when(j >= 1)
            def _():
                _wait_scatter()

            @pl.when(j + 1 < nb)
            def _():
                _fire_block(j + 1, 1 - p)

            pltpu.make_async_copy(utab.at[sstage.at[pl.ds(0, _BLK)]],
                                  rows.at[pl.ds(0, _BLK)], sem).wait()
            boff = _blk_off(j)
            for q in range(2):
                @pl.when(p == q)
                def _(_q=q):
                    for v in range(4):
                        off = pl.multiple_of(boff + v * 16, 16)
                        w = cpk[pl.ds(off, _L)]
                        tsts[_q][pl.ds(v * 16, _L)] = \
                            lax.shift_right_logical(w, 17)
                    pltpu.async_copy(rows.at[pl.ds(_q * _BLK, _BLK)],
                                     acc.at[tsts[_q]], sem3, add=True)
            return carry

        lax.fori_loop(0, nb, _blk_body, 0)

        @pl.when(nb > 0)
        def _():
            _wait_scatter()

        plsc.subcore_barrier()

        jbase = pl.multiple_of(sid * _JPT, 16)
        pltpu.sync_copy(needed.at[pl.ds(jbase, _JPT)],
                        nbuf.at[pl.ds(0, _JPT)])
        for vb in range(_JPT // 128):
            for v in range(8):
                off = pl.multiple_of(vb * 128 + v * 16, 16)
                vals = nbuf[pl.ds(off, _L)]
                sl = plsc.load_gather(map_ref, [vals])
                slotblk[pl.ds(off, _L)] = sl
            pltpu.async_copy(acc.at[slotblk.at[pl.ds(vb * 128, 128)]],
                             rows.at[pl.ds(0, 128)], sem).wait()
            pltpu.sync_copy(rows.at[pl.ds(0, 128)],
                            out.at[cid, pl.ds(jbase + vb * 128, 128)])

    return sc_fn


def _tc_body(acc_ref, w_ref, b_ref, o_ref):
    a = acc_ref[0] + acc_ref[1]
    p = lax.dot_general(a, w_ref[...], (((1,), (1,)), ((), ())),
                        preferred_element_type=jnp.float32)
    p = p + b_ref[...]
    u = p[:_B]
    t = p[_B:]
    o_ref[...] = jnp.sum(u * t, axis=1, keepdims=True)


def kernel(user_indices, item_indices, edge_index, user_table, item_table,
           W, b):
    needed = jnp.concatenate([user_indices, item_indices + _NU])
    zeros2d = jnp.zeros((_ZROWS, _D), jnp.float32)
    neg1 = jnp.full((_NTOT,), -1, jnp.int32)

    sc_fn = _make_sc_kernel()
    partials = sc_fn(edge_index, needed, user_table, item_table, zeros2d,
                     neg1)

    out = pl.pallas_call(
        _tc_body,
        out_shape=jax.ShapeDtypeStruct((_B, 1), jnp.float32),
    )(partials, W, jnp.reshape(b, (1, _D)))
    return out
